# baseline (speedup 1.0000x reference)
import functools

import jax
import jax.numpy as jnp
from jax import lax
from jax.experimental import pallas as pl
from jax.experimental.pallas import tpu as pltpu
from jax.experimental.pallas import tpu_sc as plsc

B = 16
M = 2048
F = 512
N = 16384

NC = 2
NS = 16
NW = NC * NS
HALF = M // (NW // B)

W = 128
WBITS = [64, 32, 16, 8, 4, 2, 1]
ZR = 32
ZBITS = [16, 8, 4, 2, 1]


def _sc_dense_kernel(x_hbm, params_hbm, out_hbm, buf, zbuf, pbuf, sem):
    wid = lax.axis_index("s") * NC + lax.axis_index("c")
    b = wid // 2
    h = wid % 2
    lo = h * HALF

    pltpu.sync_copy(params_hbm.at[wid], pbuf)
    v = pbuf[...]
    base = v[0]
    count = v[1]

    vlen = jnp.clip(count - lo, 0, HALF)

    @pl.loop(0, ZR)
    def _(i):
        @pl.loop(0, F, step=16)
        def _(j):
            zbuf[i, pl.ds(j, 16)] = jnp.zeros((16,), jnp.float32)

    nfull = vlen // W

    def copy_body(i, _):
        pltpu.sync_copy(x_hbm.at[pl.ds(base + lo + i * W, W)], buf)
        pltpu.sync_copy(buf, out_hbm.at[b, pl.ds(lo + i * W, W)])
        return 0

    lax.fori_loop(0, nfull, copy_body, 0)

    rem = vlen - nfull * W
    roff = lo + nfull * W
    for size in WBITS:
        @pl.when((rem & size) != 0)
        def _(size=size):
            o = roff + (rem & ~(2 * size - 1))
            pltpu.sync_copy(
                x_hbm.at[pl.ds(base + o, size)], buf.at[pl.ds(0, size)]
            )
            pltpu.sync_copy(
                buf.at[pl.ds(0, size)], out_hbm.at[b, pl.ds(o, size)]
            )

    zlen = HALF - vlen
    zoff = lo + vlen
    nz = zlen // ZR
    ztail = zlen - nz * ZR

    def z_start(i, _):
        pltpu.async_copy(
            zbuf, out_hbm.at[b, pl.ds(zoff + i * ZR, ZR)], sem
        )
        return 0

    def z_wait(i, _):
        pltpu.make_async_copy(
            zbuf, out_hbm.at[b, pl.ds(zoff + i * ZR, ZR)], sem
        ).wait()
        return 0

    lax.fori_loop(0, nz, z_start, 0)
    for size in ZBITS:
        @pl.when((ztail & size) != 0)
        def _(size=size):
            o = zoff + nz * ZR + (ztail & ~(2 * size - 1))
            pltpu.async_copy(
                zbuf.at[pl.ds(0, size)], out_hbm.at[b, pl.ds(o, size)], sem
            )
    lax.fori_loop(0, nz, z_wait, 0)
    for size in ZBITS:
        @pl.when((ztail & size) != 0)
        def _(size=size):
            o = zoff + nz * ZR + (ztail & ~(2 * size - 1))
            pltpu.make_async_copy(
                zbuf.at[pl.ds(0, size)], out_hbm.at[b, pl.ds(o, size)], sem
            ).wait()


def _sc_dense(x, params):
    mesh = plsc.VectorSubcoreMesh(core_axis_name="c", subcore_axis_name="s")
    k = pl.kernel(
        _sc_dense_kernel,
        out_type=jax.ShapeDtypeStruct((B, M, F), jnp.float32),
        mesh=mesh,
        compiler_params=pltpu.CompilerParams(use_tc_tiling_on_sc=False),
        scratch_types=[
            pltpu.VMEM((W, F), jnp.float32),
            pltpu.VMEM((ZR, F), jnp.float32),
            pltpu.VMEM((16,), jnp.int32),
            pltpu.SemaphoreType.DMA,
        ],
    )
    return k(x, params)


def _mask_body(ptr_ref, mask_ref):
    b = pl.program_id(0)
    count = ptr_ref[b + 1] - ptr_ref[b]
    row = jax.lax.broadcasted_iota(jnp.int32, (1, M), 1) < count
    mask_ref[0, 0] = jnp.broadcast_to(row, (M, M))


def _tc_mask(ptr):
    grid_spec = pltpu.PrefetchScalarGridSpec(
        num_scalar_prefetch=1,
        grid=(B,),
        in_specs=[],
        out_specs=[pl.BlockSpec((1, 1, M, M), lambda b, ptr_ref: (b, 0, 0, 0))],
    )
    return pl.pallas_call(
        _mask_body,
        grid_spec=grid_spec,
        out_shape=[jax.ShapeDtypeStruct((B, 1, M, M), jnp.bool_)],
    )(ptr)[0]


def kernel(x, batch_idx):
    ptr = jnp.searchsorted(
        batch_idx, jnp.arange(B + 1, dtype=jnp.int32), side="left"
    ).astype(jnp.int32)
    base = jnp.repeat(ptr[:B], NW // B)
    count = jnp.repeat(ptr[1:] - ptr[:B], NW // B)
    params = jnp.zeros((NW, 16), jnp.int32).at[:, 0].set(base).at[:, 1].set(count)
    dense = _sc_dense(x, params)
    mask = _tc_mask(ptr)
    return dense, mask


# speedup vs baseline: 1.7838x; 1.7838x over previous
import functools

import jax
import jax.numpy as jnp
from jax import lax
from jax.experimental import pallas as pl
from jax.experimental.pallas import tpu as pltpu
from jax.experimental.pallas import tpu_sc as plsc

B = 16
M = 2048
F = 512
N = 16384

NC = 2
NS = 16
NW = NC * NS
HALF = M // (NW // B)

W = 128
WBITS = [64, 32, 16, 8, 4, 2, 1]
ZR = 32
ZBITS = [16, 8, 4, 2, 1]


def _sc_dense_kernel(x_hbm, params_hbm, out_hbm, buf, zbuf, pbuf, sem):
    wid = lax.axis_index("s") * NC + lax.axis_index("c")
    b = wid // 2
    h = wid % 2
    lo = h * HALF

    pltpu.sync_copy(params_hbm.at[wid], pbuf)
    v = pbuf[...]
    base = v[0]
    count = v[1]

    vlen = jnp.clip(count - lo, 0, HALF)

    @pl.loop(0, ZR)
    def _(i):
        @pl.loop(0, F, step=16)
        def _(j):
            zbuf[i, pl.ds(j, 16)] = jnp.zeros((16,), jnp.float32)

    nfull = vlen // W

    def copy_body(i, _):
        pltpu.sync_copy(x_hbm.at[pl.ds(base + lo + i * W, W)], buf)
        pltpu.sync_copy(buf, out_hbm.at[b, pl.ds(lo + i * W, W)])
        return 0

    lax.fori_loop(0, nfull, copy_body, 0)

    rem = vlen - nfull * W
    roff = lo + nfull * W
    for size in WBITS:
        @pl.when((rem & size) != 0)
        def _(size=size):
            o = roff + (rem & ~(2 * size - 1))
            pltpu.sync_copy(
                x_hbm.at[pl.ds(base + o, size)], buf.at[pl.ds(0, size)]
            )
            pltpu.sync_copy(
                buf.at[pl.ds(0, size)], out_hbm.at[b, pl.ds(o, size)]
            )

    zlen = HALF - vlen
    zoff = lo + vlen
    nz = zlen // ZR
    ztail = zlen - nz * ZR

    def z_start(i, _):
        pltpu.async_copy(
            zbuf, out_hbm.at[b, pl.ds(zoff + i * ZR, ZR)], sem
        )
        return 0

    def z_wait(i, _):
        pltpu.make_async_copy(
            zbuf, out_hbm.at[b, pl.ds(zoff + i * ZR, ZR)], sem
        ).wait()
        return 0

    lax.fori_loop(0, nz, z_start, 0)
    for size in ZBITS:
        @pl.when((ztail & size) != 0)
        def _(size=size):
            o = zoff + nz * ZR + (ztail & ~(2 * size - 1))
            pltpu.async_copy(
                zbuf.at[pl.ds(0, size)], out_hbm.at[b, pl.ds(o, size)], sem
            )
    lax.fori_loop(0, nz, z_wait, 0)
    for size in ZBITS:
        @pl.when((ztail & size) != 0)
        def _(size=size):
            o = zoff + nz * ZR + (ztail & ~(2 * size - 1))
            pltpu.make_async_copy(
                zbuf.at[pl.ds(0, size)], out_hbm.at[b, pl.ds(o, size)], sem
            ).wait()


def _sc_dense(x, params):
    mesh = plsc.VectorSubcoreMesh(core_axis_name="c", subcore_axis_name="s")
    k = pl.kernel(
        _sc_dense_kernel,
        out_type=jax.ShapeDtypeStruct((B, M, F), jnp.float32),
        mesh=mesh,
        compiler_params=pltpu.CompilerParams(use_tc_tiling_on_sc=False),
        scratch_types=[
            pltpu.VMEM((W, F), jnp.float32),
            pltpu.VMEM((ZR, F), jnp.float32),
            pltpu.VMEM((16,), jnp.int32),
            pltpu.SemaphoreType.DMA,
        ],
    )
    return k(x, params)


def _dmask_body(counts_ref, dmask_ref):
    col = jax.lax.broadcasted_iota(jnp.int32, (B, M), 1)
    dmask_ref[...] = col < counts_ref[...]


def _tc_dmask(counts):
    return pl.pallas_call(
        _dmask_body,
        out_shape=[jax.ShapeDtypeStruct((B, M), jnp.bool_)],
    )(counts.reshape(B, 1))[0]


def kernel(x, batch_idx):
    counts = jnp.sum(
        batch_idx[None, :] == jnp.arange(B, dtype=jnp.int32)[:, None],
        axis=1,
        dtype=jnp.int32,
    )
    base_b = jnp.concatenate(
        [jnp.zeros((1,), jnp.int32), jnp.cumsum(counts)[: B - 1]]
    )
    base = jnp.repeat(base_b, NW // B)
    count = jnp.repeat(counts, NW // B)
    params = jnp.zeros((NW, 16), jnp.int32).at[:, 0].set(base).at[:, 1].set(count)
    dense = _sc_dense(x, params)
    dmask = _tc_dmask(counts)
    mask = jnp.broadcast_to(dmask[:, None, None, :], (B, 1, M, M))
    return dense, mask


# speedup vs baseline: 2.9627x; 1.6609x over previous
import jax
import jax.numpy as jnp
from jax import lax
from jax.experimental import pallas as pl
from jax.experimental.pallas import tpu as pltpu
from jax.experimental.pallas import tpu_sc as plsc

B = 16
M = 2048
F = 512
N = 16384

NC = 2
NS = 16
NW = NC * NS
HALF = M // (NW // B)
GPT = HALF // 8
TRG = 32
XTR = N // 8 * 4 * 8
DTR = B * M // 8 * 4 * 8

GC = 4
ZGROUPS = 8


def _iota16():
    return lax.broadcasted_iota(jnp.int32, (16,), 0)


def _build_idx(idxref, n, g0, srcbase):
    for t0 in range(0, n, 16):
        tv = _iota16() + t0
        gl = g0 + (tv >> 5)
        j = (tv >> 3) & 3
        sr = srcbase + (gl << 3) + (tv & 7)
        src = jnp.clip(sr, 0, N - 1)
        idxref[pl.ds(t0, 16)] = ((src >> 3) << 5) + (j << 3) + (src & 7)


def _sc_dense_kernel(x_hbm, params_hbm, out_hbm,
                     gbuf, zbuf, pbuf, idx128, idx64, idx32, sem):
    wid = lax.axis_index("s") * NC + lax.axis_index("c")
    b = wid // 2
    h = wid % 2
    lo = h * HALF
    dbase = b * (M // 8 * TRG) + h * (GPT * TRG)

    pltpu.sync_copy(params_hbm.at[wid], pbuf)
    v = pbuf[...]
    base = v[0]
    count = v[1]
    srcbase = base + lo

    vlen = jnp.clip(count - lo, 0, HALF)
    vfrac = vlen & 7
    ngf = vlen >> 3
    ngv = ngf + jnp.where(vfrac != 0, 1, 0)

    @pl.loop(0, ZGROUPS * TRG)
    def _(i):
        @pl.loop(0, 128, step=16)
        def _(j):
            zbuf[i, pl.ds(j, 16)] = jnp.zeros((16,), jnp.float32)

    zrows = (GPT - ngv) * TRG
    zoff = dbase + ngv * TRG
    ZB = ZGROUPS * TRG
    nz = zrows // ZB

    def z_start(i, _):
        pltpu.async_copy(zbuf, out_hbm.at[pl.ds(zoff + i * ZB, ZB)], sem)
        return 0

    lax.fori_loop(0, nz, z_start, 0)
    ztail = zrows - nz * ZB
    for size in (128, 64, 32):
        @pl.when((ztail & size) != 0)
        def _(size=size):
            o = zoff + nz * ZB + (ztail & ~(2 * size - 1))
            pltpu.async_copy(
                zbuf.at[pl.ds(0, size)], out_hbm.at[pl.ds(o, size)], sem
            )

    nfull = ngf >> 2

    def chunk_body(i, _):
        g0 = i * GC
        _build_idx(idx128, GC * TRG, g0, srcbase)
        pltpu.sync_copy(x_hbm.at[idx128], gbuf)
        pltpu.sync_copy(gbuf, out_hbm.at[pl.ds(dbase + g0 * TRG, GC * TRG)])
        return 0

    lax.fori_loop(0, nfull, chunk_body, 0)

    rem = ngf & 3
    for size, idxref in ((2, idx64), (1, idx32)):
        @pl.when((rem & size) != 0)
        def _(size=size, idxref=idxref):
            g0 = (ngf & ~3) + (rem & ~(2 * size - 1))
            _build_idx(idxref, size * TRG, g0, srcbase)
            pltpu.sync_copy(x_hbm.at[idxref], gbuf.at[pl.ds(0, size * TRG)])
            pltpu.sync_copy(
                gbuf.at[pl.ds(0, size * TRG)],
                out_hbm.at[pl.ds(dbase + g0 * TRG, size * TRG)],
            )

    @pl.when(vfrac != 0)
    def _():
        _build_idx(idx32, TRG, ngf, srcbase)
        pltpu.sync_copy(x_hbm.at[idx32], gbuf.at[pl.ds(0, TRG)])

        def fix_r(r, _):
            for j in range(4):
                @pl.loop(0, 128, step=16)
                def _(c, j=j, r=r):
                    gbuf[(j << 3) + r, pl.ds(c, 16)] = jnp.zeros(
                        (16,), jnp.float32
                    )
            return 0

        lax.fori_loop(vfrac, 8, fix_r, 0)
        pltpu.sync_copy(
            gbuf.at[pl.ds(0, TRG)],
            out_hbm.at[pl.ds(dbase + ngf * TRG, TRG)],
        )

    def z_wait(i, _):
        pltpu.make_async_copy(
            zbuf, out_hbm.at[pl.ds(zoff + i * ZB, ZB)], sem
        ).wait()
        return 0

    lax.fori_loop(0, nz, z_wait, 0)
    for size in (128, 64, 32):
        @pl.when((ztail & size) != 0)
        def _(size=size):
            o = zoff + nz * ZB + (ztail & ~(2 * size - 1))
            pltpu.make_async_copy(
                zbuf.at[pl.ds(0, size)], out_hbm.at[pl.ds(o, size)], sem
            ).wait()


def _sc_dense(x, params):
    x2 = (
        x.reshape(N // 8, 8, 4, 128)
        .transpose(0, 2, 1, 3)
        .reshape(XTR, 128)
    )
    mesh = plsc.VectorSubcoreMesh(core_axis_name="c", subcore_axis_name="s")
    k = pl.kernel(
        _sc_dense_kernel,
        out_type=jax.ShapeDtypeStruct((DTR, 128), jnp.float32),
        mesh=mesh,
        compiler_params=pltpu.CompilerParams(use_tc_tiling_on_sc=False),
        scratch_types=[
            pltpu.VMEM((GC * TRG, 128), jnp.float32),
            pltpu.VMEM((ZGROUPS * TRG, 128), jnp.float32),
            pltpu.VMEM((16,), jnp.int32),
            pltpu.VMEM((GC * TRG,), jnp.int32),
            pltpu.VMEM((2 * TRG,), jnp.int32),
            pltpu.VMEM((TRG,), jnp.int32),
            pltpu.SemaphoreType.DMA,
        ],
    )
    out2 = k(x2, params)
    return (
        out2.reshape(B, M // 8, 4, 8, 128)
        .transpose(0, 1, 3, 2, 4)
        .reshape(B, M, F)
    )


def _dmask_body(counts_ref, dmask_ref):
    col = jax.lax.broadcasted_iota(jnp.int32, (B, M), 1)
    dmask_ref[...] = col < counts_ref[...]


def _tc_dmask(counts):
    return pl.pallas_call(
        _dmask_body,
        out_shape=[jax.ShapeDtypeStruct((B, M), jnp.bool_)],
    )(counts.reshape(B, 1))[0]


def kernel(x, batch_idx):
    counts = jnp.sum(
        batch_idx[None, :] == jnp.arange(B, dtype=jnp.int32)[:, None],
        axis=1,
        dtype=jnp.int32,
    )
    base_b = jnp.concatenate(
        [jnp.zeros((1,), jnp.int32), jnp.cumsum(counts)[: B - 1]]
    )
    base = jnp.repeat(base_b, NW // B)
    count = jnp.repeat(counts, NW // B)
    params = jnp.zeros((NW, 16), jnp.int32).at[:, 0].set(base).at[:, 1].set(count)
    dense = _sc_dense(x, params)
    dmask = _tc_dmask(counts)
    mask = jnp.broadcast_to(dmask[:, None, None, :], (B, 1, M, M))
    return dense, mask
